# baseline (speedup 1.0000x reference)
import functools
import math

import jax
import jax.numpy as jnp
from jax import lax
from jax.experimental import pallas as pl
from jax.experimental.pallas import tpu as pltpu
from jax.experimental.pallas import tpu_sc as plsc

EMB = 32
ROWS = 4096
COLS = 200
V = 1000000
NC, NS = 2, 16
NW = NC * NS
SCALE = math.sqrt(float(EMB))

CH_A = 1024
N_UNIF = 30
N_EXTRA = 16
V_REM0 = 999424
V_TAIL0 = 999936
IBLK = ROWS // NW
JBLK = 8
NBLK_B = COLS // JBLK

_mesh = plsc.VectorSubcoreMesh(core_axis_name="c", subcore_axis_name="s")


@functools.partial(
    pl.kernel,
    mesh=_mesh,
    out_type=jax.ShapeDtypeStruct((V // 32, 8, 128), jnp.float32),
    scratch_types=[
        pltpu.VMEM((4, 8, CH_A), jnp.float32),
        pltpu.VMEM((4, 8, CH_A), jnp.float32),
        pltpu.VMEM((32, 8, 128), jnp.float32),
        pltpu.SemaphoreType.DMA,
        pltpu.SemaphoreType.DMA,
        pltpu.SemaphoreType.DMA,
    ],
    compiler_params=pltpu.CompilerParams(use_tc_tiling_on_sc=True,
                                         needs_layout_passes=False),
)
def _format_table(t4, tail, out2, sb0, sb1, obuf, si0, si1, so):
    w = lax.axis_index("s") * NC + lax.axis_index("c")
    sb = (sb0, sb1)
    sis = (si0, si1)
    iota32 = lax.iota(jnp.int32, 16) * 32

    def issue_in(v0, b, nrows):
        return [pltpu.async_copy(t4.at[tc, :, pl.ds(v0, nrows)],
                                 sb[b].at[tc, :, pl.ds(0, nrows)], sis[b])
                for tc in range(4)]

    def transpose_block(src, nm):
        @plsc.parallel_loop(0, nm, step=1)
        def _(m):
            fbase = iota32 + m * 512
            b_vec = lax.shift_right_logical(fbase, 10)
            s_vec = lax.shift_right_logical(fbase, 7) & 7
            lbase = fbase & 127
            vt16 = m * 16
            for c in range(32):
                vvec = src[c // 8, c % 8, pl.ds(vt16, 16)] * SCALE
                plsc.store_scatter(obuf, [b_vec, s_vec, lbase + c], vvec)

    obuf3 = obuf

    in_d = [None, None]
    out_d = None
    in_d[0] = issue_in(w * N_UNIF * CH_A, 0, CH_A)
    for k in range(N_UNIF):
        bb = k & 1
        nb = bb ^ 1
        if k + 1 < N_UNIF:
            in_d[nb] = issue_in((w * N_UNIF + k + 1) * CH_A, nb, CH_A)
        for d in in_d[bb]:
            d.wait()
        if out_d is not None:
            out_d.wait()
        transpose_block(sb[bb], 64)
        out_d = pltpu.async_copy(
            obuf3, out2.at[pl.ds((w * N_UNIF + k) * 32, 32)], so)
    out_d.wait()

    @pl.when(w < N_EXTRA)
    def _():
        v0 = (NW * N_UNIF + w) * CH_A
        for d in issue_in(v0, 0, CH_A):
            d.wait()
        transpose_block(sb[0], 64)
        pltpu.async_copy(obuf3, out2.at[pl.ds(v0 // 32, 32)], so).wait()

    @pl.when(w == N_EXTRA)
    def _():
        for d in issue_in(V_REM0, 0, 512):
            d.wait()
        transpose_block(sb[0], 32)
        pltpu.async_copy(obuf3.at[pl.ds(0, 16)],
                         out2.at[pl.ds(V_REM0 // 32, 16)], so).wait()

    @pl.when(w == N_EXTRA + 1)
    def _():
        pltpu.sync_copy(tail, out2.at[pl.ds(V_TAIL0 // 32, 2)])


@functools.partial(
    pl.kernel,
    mesh=_mesh,
    out_type=jax.ShapeDtypeStruct((COLS, 4, NW, 8, 128), jnp.float32),
    scratch_types=[
        pltpu.VMEM((JBLK * IBLK,), jnp.int32),
        pltpu.VMEM((JBLK * IBLK,), jnp.int32),
        pltpu.VMEM((JBLK * IBLK, EMB), jnp.float32),
        pltpu.VMEM((JBLK * IBLK, EMB), jnp.float32),
        pltpu.VMEM((JBLK, 1, 1, 8, 128), jnp.float32),
        pltpu.VMEM((JBLK, 1, 1, 8, 128), jnp.float32),
        pltpu.VMEM((JBLK, 1, 1, 8, 128), jnp.float32),
        pltpu.VMEM((JBLK, 1, 1, 8, 128), jnp.float32),
        pltpu.SemaphoreType.DMA,
        pltpu.SemaphoreType.DMA,
        pltpu.SemaphoreType.DMA,
        pltpu.SemaphoreType.DMA,
        pltpu.SemaphoreType.DMA,
    ],
    compiler_params=pltpu.CompilerParams(use_tc_tiling_on_sc=False,
                                         needs_layout_passes=False),
)
def _gather(tokT, table_rm, out5, ix0, ix1, rw0, rw1, ob0, ob1, ob2, ob3,
            gi0, gi1, gg0, gg1, go):
    w = lax.axis_index("s") * NC + lax.axis_index("c")
    i0 = w * IBLK
    obufs = (ob0, ob1, ob2, ob3)
    iota = lax.iota(jnp.int32, 16)
    c_splats = [jnp.full((16,), c, jnp.int32) for c in range(EMB)]

    def issue_idx(j0, ixb, gib):
        for jj in range(JBLK):
            pltpu.async_copy(tokT.at[j0 + jj, pl.ds(i0, IBLK)],
                             ixb.at[pl.ds(jj * IBLK, IBLK)], gib)

    def drain_idx(ixb, gib):
        pltpu.make_async_copy(tokT.at[0, pl.ds(0, JBLK * IBLK)], ixb, gib).wait()

    def issue_gather(ixb, rwb, ggb):
        pltpu.async_copy(table_rm.at[ixb], rwb, ggb)

    def drain_gather(rwb, ggb):
        pltpu.make_async_copy(table_rm.at[pl.ds(0, JBLK * IBLK)], rwb,
                              ggb).wait()

    def issue_out(j0):
        for cb in range(4):
            pltpu.async_copy(obufs[cb],
                             out5.at[pl.ds(j0, JBLK), pl.ds(cb, 1),
                                     pl.ds(w, 1)], go)

    def drain_out():
        for cb in range(4):
            pltpu.make_async_copy(
                out5.at[pl.ds(0, JBLK), pl.ds(cb, 1), pl.ds(0, 1)],
                obufs[cb], go).wait()

    def transpose_rows(src):
        @plsc.parallel_loop(0, JBLK * IBLK // 16, step=1, unroll=4)
        def _(u):
            jj = lax.shift_right_logical(u, 3)
            ii0 = (u & 7) * 16
            r_vec = iota + (jj * IBLK + ii0)
            for c in range(EMB):
                vvec = plsc.load_gather(src, [r_vec, c_splats[c]]) * SCALE
                obufs[c >> 3][jj, 0, 0, c & 7, pl.ds(ii0, 16)] = vvec

    issue_idx(0, ix0, gi0)
    drain_idx(ix0, gi0)
    issue_gather(ix0, rw0, gg0)
    issue_idx(JBLK, ix1, gi1)

    def body(t, carry):
        jA = 2 * t * JBLK
        jB = jA + JBLK
        drain_idx(ix1, gi1)
        drain_gather(rw0, gg0)
        issue_idx(jB + JBLK, ix0, gi0)
        issue_gather(ix1, rw1, gg1)

        @pl.when(t > 0)
        def _():
            drain_out()
        transpose_rows(rw0)
        issue_out(jA)
        drain_idx(ix0, gi0)
        drain_gather(rw1, gg1)

        @pl.when(t < NBLK_B // 2 - 1)
        def _():
            issue_idx(jB + 2 * JBLK, ix1, gi1)
        issue_gather(ix0, rw0, gg0)
        drain_out()
        transpose_rows(rw1)
        issue_out(jB)
        return carry

    lax.fori_loop(0, NBLK_B // 2, body, 0)

    jL = (NBLK_B - 1) * JBLK
    drain_gather(rw0, gg0)
    drain_out()
    transpose_rows(rw0)
    issue_out(jL)
    drain_out()


def kernel(tokens, table):
    out5 = _gather(tokens.T, table)
    out_t = out5.transpose(0, 1, 3, 2, 4).reshape(COLS, EMB, ROWS)
    return jnp.transpose(out_t, (2, 0, 1))


# speedup vs baseline: 1.4992x; 1.4992x over previous
import functools
import math

import jax
import jax.numpy as jnp
from jax import lax
from jax.experimental import pallas as pl
from jax.experimental.pallas import tpu as pltpu
from jax.experimental.pallas import tpu_sc as plsc

EMB = 32
ROWS = 4096
COLS = 200
V = 1000000
NC, NS = 2, 16
NW = NC * NS
SCALE = math.sqrt(float(EMB))

CH_A = 1024
N_UNIF = 30
N_EXTRA = 16
V_REM0 = 999424
V_TAIL0 = 999936
IBLK = ROWS // NW
JBLK = 8
NBLK_B = COLS // JBLK

_mesh = plsc.VectorSubcoreMesh(core_axis_name="c", subcore_axis_name="s")


@functools.partial(
    pl.kernel,
    mesh=_mesh,
    out_type=jax.ShapeDtypeStruct((V // 32, 8, 128), jnp.float32),
    scratch_types=[
        pltpu.VMEM((4, 8, CH_A), jnp.float32),
        pltpu.VMEM((4, 8, CH_A), jnp.float32),
        pltpu.VMEM((32, 8, 128), jnp.float32),
        pltpu.SemaphoreType.DMA,
        pltpu.SemaphoreType.DMA,
        pltpu.SemaphoreType.DMA,
    ],
    compiler_params=pltpu.CompilerParams(use_tc_tiling_on_sc=True,
                                         needs_layout_passes=False),
)
def _format_table(t4, tail, out2, sb0, sb1, obuf, si0, si1, so):
    w = lax.axis_index("s") * NC + lax.axis_index("c")
    sb = (sb0, sb1)
    sis = (si0, si1)
    iota32 = lax.iota(jnp.int32, 16) * 32

    def issue_in(v0, b, nrows):
        return [pltpu.async_copy(t4.at[tc, :, pl.ds(v0, nrows)],
                                 sb[b].at[tc, :, pl.ds(0, nrows)], sis[b])
                for tc in range(4)]

    def transpose_block(src, nm):
        @plsc.parallel_loop(0, nm, step=1)
        def _(m):
            fbase = iota32 + m * 512
            b_vec = lax.shift_right_logical(fbase, 10)
            s_vec = lax.shift_right_logical(fbase, 7) & 7
            lbase = fbase & 127
            vt16 = m * 16
            for c in range(32):
                vvec = src[c // 8, c % 8, pl.ds(vt16, 16)] * SCALE
                plsc.store_scatter(obuf, [b_vec, s_vec, lbase + c], vvec)

    obuf3 = obuf

    in_d = [None, None]
    out_d = None
    in_d[0] = issue_in(w * N_UNIF * CH_A, 0, CH_A)
    for k in range(N_UNIF):
        bb = k & 1
        nb = bb ^ 1
        if k + 1 < N_UNIF:
            in_d[nb] = issue_in((w * N_UNIF + k + 1) * CH_A, nb, CH_A)
        for d in in_d[bb]:
            d.wait()
        if out_d is not None:
            out_d.wait()
        transpose_block(sb[bb], 64)
        out_d = pltpu.async_copy(
            obuf3, out2.at[pl.ds((w * N_UNIF + k) * 32, 32)], so)
    out_d.wait()

    @pl.when(w < N_EXTRA)
    def _():
        v0 = (NW * N_UNIF + w) * CH_A
        for d in issue_in(v0, 0, CH_A):
            d.wait()
        transpose_block(sb[0], 64)
        pltpu.async_copy(obuf3, out2.at[pl.ds(v0 // 32, 32)], so).wait()

    @pl.when(w == N_EXTRA)
    def _():
        for d in issue_in(V_REM0, 0, 512):
            d.wait()
        transpose_block(sb[0], 32)
        pltpu.async_copy(obuf3.at[pl.ds(0, 16)],
                         out2.at[pl.ds(V_REM0 // 32, 16)], so).wait()

    @pl.when(w == N_EXTRA + 1)
    def _():
        pltpu.sync_copy(tail, out2.at[pl.ds(V_TAIL0 // 32, 2)])


@functools.partial(
    pl.kernel,
    mesh=_mesh,
    out_type=jax.ShapeDtypeStruct((COLS, 4, NW, 8, 128), jnp.float32),
    scratch_types=[
        pltpu.VMEM((JBLK * IBLK,), jnp.int32),
        pltpu.VMEM((JBLK * IBLK,), jnp.int32),
        pltpu.VMEM((JBLK * IBLK, EMB), jnp.float32),
        pltpu.VMEM((JBLK * IBLK, EMB), jnp.float32),
        pltpu.VMEM((4, JBLK, 1, 1, 8, 129), jnp.float32),
        pltpu.SemaphoreType.DMA,
        pltpu.SemaphoreType.DMA,
        pltpu.SemaphoreType.DMA,
        pltpu.SemaphoreType.DMA,
        pltpu.SemaphoreType.DMA,
    ],
    compiler_params=pltpu.CompilerParams(use_tc_tiling_on_sc=False,
                                         needs_layout_passes=False),
)
def _gather(tokT, table_rm, out5, ix0, ix1, rw0, rw1, obuf,
            gi0, gi1, gg0, gg1, go):
    w = lax.axis_index("s") * NC + lax.axis_index("c")
    i0 = w * IBLK
    iota = lax.iota(jnp.int32, 16)
    zeros16 = jnp.zeros((16,), jnp.int32)
    cb_vecs = [lax.shift_right_logical(h * 16 + iota, 3) for h in range(2)]
    cl_vecs = [(h * 16 + iota) & 7 for h in range(2)]

    def issue_idx(j0, ixb, gib):
        for jj in range(JBLK):
            pltpu.async_copy(tokT.at[j0 + jj, pl.ds(i0, IBLK)],
                             ixb.at[pl.ds(jj * IBLK, IBLK)], gib)

    def drain_idx(ixb, gib):
        pltpu.make_async_copy(tokT.at[0, pl.ds(0, JBLK * IBLK)], ixb, gib).wait()

    def issue_gather(ixb, rwb, ggb):
        pltpu.async_copy(table_rm.at[ixb], rwb, ggb)

    def drain_gather(rwb, ggb):
        pltpu.make_async_copy(table_rm.at[pl.ds(0, JBLK * IBLK)], rwb,
                              ggb).wait()

    def issue_out(j0):
        for cb in range(4):
            pltpu.async_copy(obuf.at[cb, :, :, :, :, pl.ds(0, 128)],
                             out5.at[pl.ds(j0, JBLK), pl.ds(cb, 1),
                                     pl.ds(w, 1)], go)

    def drain_out():
        for cb in range(4):
            pltpu.make_async_copy(
                out5.at[pl.ds(0, JBLK), pl.ds(cb, 1), pl.ds(0, 1)],
                obuf.at[cb, :, :, :, :, pl.ds(0, 128)], go).wait()

    def transpose_rows(src):
        @plsc.parallel_loop(0, JBLK * IBLK, step=1)
        def _(r):
            jj = lax.shift_right_logical(r, 7)
            ii = r & 127
            jj_s = jnp.broadcast_to(jj, (16,))
            ii_s = jnp.broadcast_to(ii, (16,))
            for h in range(2):
                vvec = src[r, pl.ds(h * 16, 16)] * SCALE
                plsc.store_scatter(
                    obuf, [cb_vecs[h], jj_s, zeros16, zeros16, cl_vecs[h],
                           ii_s], vvec)

    issue_idx(0, ix0, gi0)
    drain_idx(ix0, gi0)
    issue_gather(ix0, rw0, gg0)
    issue_idx(JBLK, ix1, gi1)

    def body(t, carry):
        jA = 2 * t * JBLK
        jB = jA + JBLK
        drain_idx(ix1, gi1)
        drain_gather(rw0, gg0)
        issue_idx(jB + JBLK, ix0, gi0)
        issue_gather(ix1, rw1, gg1)

        @pl.when(t > 0)
        def _():
            drain_out()
        transpose_rows(rw0)
        issue_out(jA)
        drain_idx(ix0, gi0)
        drain_gather(rw1, gg1)

        @pl.when(t < NBLK_B // 2 - 1)
        def _():
            issue_idx(jB + 2 * JBLK, ix1, gi1)
        issue_gather(ix0, rw0, gg0)
        drain_out()
        transpose_rows(rw1)
        issue_out(jB)
        return carry

    lax.fori_loop(0, NBLK_B // 2, body, 0)

    jL = (NBLK_B - 1) * JBLK
    drain_gather(rw0, gg0)
    drain_out()
    transpose_rows(rw0)
    issue_out(jL)
    drain_out()


def kernel(tokens, table):
    out5 = _gather(tokens.T, table)
    out_t = out5.transpose(0, 1, 3, 2, 4).reshape(COLS, EMB, ROWS)
    return jnp.transpose(out_t, (2, 0, 1))
